# baseline (speedup 1.0000x reference)
import functools

import jax
import jax.numpy as jnp
from jax import lax
from jax.experimental import pallas as pl
from jax.experimental.pallas import tpu as pltpu
from jax.experimental.pallas import tpu_sc as plsc

D_MODEL = 64
SCALE = 8.0

_INFO = plsc.get_sparse_core_info()
NC = _INFO.num_cores
NS = _INFO.num_subcores
NW = NC * NS
LANES = _INFO.num_lanes

CHUNK = 128
NBUF = 6
HALF = NBUF // 2
NO = 2


def _make_kernel(n_idx: int):
  assert n_idx % (NW * CHUNK) == 0
  per_w = n_idx // NW
  n_chunks = per_w // CHUNK
  assert n_chunks > NBUF

  mesh = plsc.VectorSubcoreMesh(core_axis_name="c", subcore_axis_name="s")

  @functools.partial(
      pl.kernel,
      out_type=jax.ShapeDtypeStruct((n_idx, 2 * D_MODEL), jnp.float32),
      mesh=mesh,
      scratch_types=[
          pltpu.VMEM((n_chunks, CHUNK), jnp.int32),
          pltpu.VMEM((NBUF, CHUNK, D_MODEL), jnp.float32),
          pltpu.VMEM((NO, CHUNK, 2 * D_MODEL), jnp.float32),
          pltpu.SemaphoreType.DMA,
          pltpu.SemaphoreType.DMA,
      ],
      compiler_params=pltpu.CompilerParams(use_tc_tiling_on_sc=False),
  )
  def emb_kernel(idx_hbm, table_hbm, out_hbm, idx_v, rows_v, ot_v, gsem,
                 psem):
    wid = lax.axis_index("s") * NC + lax.axis_index("c")
    base = wid * per_w
    pltpu.sync_copy(idx_hbm.at[wid], idx_v)

    def fire_gather(j):
      pltpu.async_copy(table_hbm.at[idx_v.at[j]], rows_v.at[j % NBUF], gsem)

    def wait_gather():
      pltpu.make_async_copy(table_hbm.at[pl.ds(0, CHUNK)], rows_v.at[0],
                            gsem).wait()

    def wait_put():
      pltpu.make_async_copy(ot_v.at[0], out_hbm.at[pl.ds(0, CHUNK)],
                            psem).wait()

    for j in range(HALF):
      fire_gather(j)

    @pl.loop(0, n_chunks)
    def _chunk(j):
      bi = j % NBUF
      o = j % NO
      wait_gather()

      @pl.when(j + HALF < n_chunks)
      def _():
        fire_gather(j + HALF)

      @pl.when(j >= NO)
      def _():
        wait_put()

      @pl.loop(0, CHUNK, unroll=4)
      def _row(r):
        for c in range(D_MODEL // LANES):
          sl = pl.ds(c * LANES, LANES)
          ot_v[o, r, sl] = rows_v[bi, r, sl] * SCALE

      pltpu.async_copy(
          ot_v.at[o], out_hbm.at[pl.ds(base + j * CHUNK, CHUNK)], psem
      )

    @pl.loop(0, NO)
    def _drain(_):
      wait_put()

  return emb_kernel


def kernel(x, table):
  b, s = x.shape
  n_idx = b * s
  idx = x.reshape(NW, n_idx // (NW * CHUNK), CHUNK).astype(jnp.int32)
  out = _make_kernel(n_idx)(idx, table)
  return out[:, :D_MODEL].reshape(b, s, D_MODEL)


# speedup vs baseline: 1.0412x; 1.0412x over previous
import functools

import jax
import jax.numpy as jnp
from jax import lax
from jax.experimental import pallas as pl
from jax.experimental.pallas import tpu as pltpu
from jax.experimental.pallas import tpu_sc as plsc

D_MODEL = 64
SCALE = 8.0

_INFO = plsc.get_sparse_core_info()
NC = _INFO.num_cores
NS = _INFO.num_subcores
NW = NC * NS
LANES = _INFO.num_lanes

CHUNK = 128
NBUF = 8
HALF = NBUF // 2
NO = 4


def _make_kernel(n_idx: int):
  assert n_idx % (NW * CHUNK) == 0
  per_w = n_idx // NW
  n_chunks = per_w // CHUNK
  assert n_chunks > NBUF

  mesh = plsc.VectorSubcoreMesh(core_axis_name="c", subcore_axis_name="s")

  @functools.partial(
      pl.kernel,
      out_type=jax.ShapeDtypeStruct((n_idx, 2 * D_MODEL), jnp.float32),
      mesh=mesh,
      scratch_types=[
          pltpu.VMEM((n_chunks, CHUNK), jnp.int32),
          pltpu.VMEM((NBUF, CHUNK, D_MODEL), jnp.float32),
          pltpu.VMEM((NO, CHUNK, D_MODEL), jnp.float32),
          pltpu.SemaphoreType.DMA,
          pltpu.SemaphoreType.DMA,
      ],
      compiler_params=pltpu.CompilerParams(use_tc_tiling_on_sc=False),
  )
  def emb_kernel(idx_hbm, table_hbm, out_hbm, idx_v, rows_v, ot_v, gsem,
                 psem):
    wid = lax.axis_index("s") * NC + lax.axis_index("c")
    base = wid * per_w
    pltpu.sync_copy(idx_hbm.at[wid], idx_v)

    def fire_gather(j):
      pltpu.async_copy(table_hbm.at[idx_v.at[j]], rows_v.at[j % NBUF], gsem)

    def wait_gather():
      pltpu.make_async_copy(table_hbm.at[pl.ds(0, CHUNK)], rows_v.at[0],
                            gsem).wait()

    def wait_put():
      pltpu.make_async_copy(
          ot_v.at[0], out_hbm.at[pl.ds(0, CHUNK), pl.ds(0, D_MODEL)],
          psem).wait()

    for j in range(HALF):
      fire_gather(j)

    @pl.loop(0, n_chunks)
    def _chunk(j):
      bi = j % NBUF
      o = j % NO
      wait_gather()

      @pl.when(j + HALF < n_chunks)
      def _():
        fire_gather(j + HALF)

      @pl.when(j >= NO)
      def _():
        wait_put()

      @pl.loop(0, CHUNK, unroll=4)
      def _row(r):
        for c in range(D_MODEL // LANES):
          sl = pl.ds(c * LANES, LANES)
          ot_v[o, r, sl] = rows_v[bi, r, sl] * SCALE

      pltpu.async_copy(
          ot_v.at[o],
          out_hbm.at[pl.ds(base + j * CHUNK, CHUNK), pl.ds(0, D_MODEL)],
          psem,
      )

    @pl.loop(0, NO)
    def _drain(_):
      wait_put()

  return emb_kernel


def kernel(x, table):
  b, s = x.shape
  n_idx = b * s
  idx = x.reshape(NW, n_idx // (NW * CHUNK), CHUNK).astype(jnp.int32)
  out = _make_kernel(n_idx)(idx, table)
  return out[:, :D_MODEL].reshape(b, s, D_MODEL)


# speedup vs baseline: 1.4153x; 1.3593x over previous
import functools

import jax
import jax.numpy as jnp
from jax import lax
from jax.experimental import pallas as pl
from jax.experimental.pallas import tpu as pltpu
from jax.experimental.pallas import tpu_sc as plsc

D_MODEL = 64
SCALE = 8.0

_INFO = plsc.get_sparse_core_info()
NC = _INFO.num_cores
NS = _INFO.num_subcores
NW = NC * NS
LANES = _INFO.num_lanes

CHUNK = 128
NBUF = 8
HALF = NBUF // 2
NO = 4


def _make_kernel(n_idx: int):
  assert n_idx % (NW * CHUNK) == 0
  per_w = n_idx // NW
  n_chunks = per_w // CHUNK
  assert n_chunks > NBUF

  mesh = plsc.VectorSubcoreMesh(core_axis_name="c", subcore_axis_name="s")

  @functools.partial(
      pl.kernel,
      out_type=jax.ShapeDtypeStruct((n_idx, 2 * D_MODEL), jnp.float32),
      mesh=mesh,
      scratch_types=[
          pltpu.VMEM((n_chunks, CHUNK), jnp.int32),
          pltpu.VMEM((NBUF, CHUNK, D_MODEL), jnp.float32),
          pltpu.SemaphoreType.DMA,
          pltpu.SemaphoreType.DMA,
      ],
      compiler_params=pltpu.CompilerParams(use_tc_tiling_on_sc=False),
  )
  def emb_kernel(idx_hbm, table_hbm, out_hbm, idx_v, rows_v, gsem, psem):
    wid = lax.axis_index("s") * NC + lax.axis_index("c")
    base = wid * per_w
    pltpu.sync_copy(idx_hbm.at[wid], idx_v)

    def fire_gather(j):
      pltpu.async_copy(table_hbm.at[idx_v.at[j]], rows_v.at[j % NBUF], gsem)

    def wait_gather():
      pltpu.make_async_copy(table_hbm.at[pl.ds(0, CHUNK)], rows_v.at[0],
                            gsem).wait()

    def wait_put():
      pltpu.make_async_copy(
          rows_v.at[0], out_hbm.at[pl.ds(0, CHUNK), pl.ds(0, D_MODEL)],
          psem).wait()

    for j in range(HALF):
      fire_gather(j)

    @pl.loop(0, n_chunks)
    def _chunk(j):
      bi = j % NBUF
      wait_gather()

      @pl.loop(0, CHUNK, unroll=4)
      def _row(r):
        for c in range(D_MODEL // LANES):
          sl = pl.ds(c * LANES, LANES)
          rows_v[bi, r, sl] = rows_v[bi, r, sl] * SCALE

      pltpu.async_copy(
          rows_v.at[bi],
          out_hbm.at[pl.ds(base + j * CHUNK, CHUNK), pl.ds(0, D_MODEL)],
          psem,
      )

      jn = j + HALF

      @pl.when(jn < n_chunks)
      def _():
        @pl.when(j >= HALF)
        def _():
          wait_put()
        fire_gather(jn)

    @pl.loop(0, NBUF)
    def _drain(_):
      wait_put()

  return emb_kernel


def kernel(x, table):
  b, s = x.shape
  n_idx = b * s
  idx = x.reshape(NW, n_idx // (NW * CHUNK), CHUNK).astype(jnp.int32)
  out = _make_kernel(n_idx)(idx, table)
  return out[:, :D_MODEL].reshape(b, s, D_MODEL)
